# Initial kernel scaffold; baseline (speedup 1.0000x reference)
import jax
import jax.numpy as jnp
from jax.experimental import pallas as pl


def kernel(q_idx, embed_para):
    raise NotImplementedError("write your pallas kernel here")


# speedup vs baseline: 1.6462x; 1.6462x over previous
import functools

import jax
import jax.numpy as jnp
from jax import lax
from jax.experimental import pallas as pl
from jax.experimental.pallas import tpu as pltpu
from jax.experimental.pallas import tpu_sc as plsc

VOCAB = 1000000
EMBED_DIM = 32
BATCH = 16384
N_FIELDS = 26

_B = BATCH * N_FIELDS
_IDX_MINOR = 128
_N_IDX_ROWS = _B // _IDX_MINOR
_G = 8


def _make_kernel():
    info = plsc.get_sparse_core_info()
    nc, ns = info.num_cores, info.num_subcores
    nw = nc * ns
    rows_pw = _N_IDX_ROWS // nw
    n_groups = rows_pw // _G

    mesh = plsc.VectorSubcoreMesh(core_axis_name="c", subcore_axis_name="s")

    @functools.partial(
        pl.kernel,
        mesh=mesh,
        compiler_params=pltpu.CompilerParams(use_tc_tiling_on_sc=False),
        out_type=jax.ShapeDtypeStruct((_B, EMBED_DIM), jnp.float32),
        scratch_types=[
            pltpu.VMEM((rows_pw, _IDX_MINOR), jnp.int32),
            pltpu.VMEM((rows_pw, _IDX_MINOR), jnp.int32),
            pltpu.VMEM((rows_pw + 16,), jnp.int32),
            pltpu.VMEM((_G * _IDX_MINOR, EMBED_DIM), jnp.float32),
            pltpu.SemaphoreType.DMA,
        ],
    )
    def emb_kernel(idx_hbm, flags_hbm, table_hbm, out_hbm,
                   idx_v, cidx_v, flags_v, rows_v, sem):
        wid = lax.axis_index("s") * nc + lax.axis_index("c")
        row0 = wid * rows_pw
        out0 = row0 * _IDX_MINOR

        pltpu.sync_copy(idx_hbm.at[pl.ds(row0, rows_pw)], idx_v)
        flags_v[pl.ds(rows_pw, 16)] = jnp.zeros((16,), jnp.int32)
        pltpu.sync_copy(flags_hbm.at[pl.ds(row0, rows_pw)],
                        flags_v.at[pl.ds(0, rows_pw)])

        zrow = jnp.zeros((16,), jnp.float32)
        lane = lax.iota(jnp.int32, 16)

        def group_body(g, _):
            copies = []
            for j in range(_G):
                r = g * _G + j
                for c in range(_IDX_MINOR // 16):
                    v = idx_v[r, pl.ds(c * 16, 16)]
                    cidx_v[r, pl.ds(c * 16, 16)] = jnp.maximum(v - 1, 0)
                copies.append(pltpu.async_copy(
                    table_hbm.at[cidx_v.at[r]],
                    rows_v.at[pl.ds(j * _IDX_MINOR, _IDX_MINOR)],
                    sem,
                ))
            for cp in copies:
                cp.wait()

            fg = flags_v[pl.ds(g * _G, 16)]
            for j in range(_G):
                r = g * _G + j

                @pl.when(fg[j] != 0)
                def _fix(r=r, j=j):
                    def grp_body(c, _):
                        m = jnp.minimum(idx_v[r, pl.ds(c * 16, 16)], 1)
                        rbase = j * _IDX_MINOR + c * 16
                        for l in range(16):
                            @pl.when(m[l] == 0)
                            def _zero(l=l):
                                rows_v[rbase + l, pl.ds(0, 16)] = zrow
                                rows_v[rbase + l, pl.ds(16, 16)] = zrow
                        return ()
                    lax.fori_loop(0, _IDX_MINOR // 16, grp_body, ())

            pltpu.sync_copy(
                rows_v,
                out_hbm.at[pl.ds(out0 + g * _G * _IDX_MINOR, _G * _IDX_MINOR)],
            )
            return ()

        lax.fori_loop(0, n_groups, group_body, ())

    return emb_kernel


def kernel(q_idx, embed_para):
    idx2d = q_idx.astype(jnp.int32).reshape(_N_IDX_ROWS, _IDX_MINOR)
    flags = (idx2d == 0).any(axis=1).astype(jnp.int32)
    out = _make_kernel()(idx2d, flags, embed_para)
    return out.reshape(BATCH, N_FIELDS, EMBED_DIM)
